# baseline (speedup 1.0000x reference)
import jax
import jax.numpy as jnp
from jax.experimental import pallas as pl
from jax.experimental.pallas import tpu as pltpu

_B = 16384
_P = 1024
_D = 128
_BZ = 1024
_NB = _B // _BZ
_REG1 = 0.05
_REG2 = 0.05


def _compute(z_ref, pm2_ref, cross_ref, z2_ref):
    zb = z_ref[:]
    z2_ref[:] = jnp.sum(zb * zb, axis=1, keepdims=True)
    cross_ref[:] = jax.lax.dot_general(
        zb.astype(jnp.bfloat16), pm2_ref[:],
        (((1,), (1,)), ((), ())),
        preferred_element_type=jnp.float32)


def _epilogue(i, cross_ref, z2_ref, p2_ref, colmin_ref, rowacc_ref):
    z2 = z2_ref[:]
    t = cross_ref[:] + p2_ref[:]
    rowmin = jnp.min(t, axis=1, keepdims=True)
    rowpart = jnp.sqrt(jnp.maximum(rowmin + z2, 0.0))
    colpart = jnp.min(t + z2, axis=0, keepdims=True)

    @pl.when(i == 1)
    def _init():
        rowacc_ref[:] = rowpart
        colmin_ref[:] = colpart

    @pl.when(i > 1)
    def _accum():
        rowacc_ref[:] = rowacc_ref[:] + rowpart
        colmin_ref[:] = jnp.minimum(colmin_ref[:], colpart)


def _loss_body(z_ref, p_ref, out_ref, pm2_ref, p2_ref, cross_a, cross_b,
               z2_a, z2_b, colmin_ref, rowacc_ref):
    i = pl.program_id(0)

    @pl.when(i == 0)
    def _first():
        p = p_ref[:]
        pm2_ref[:] = (-2.0 * p).astype(jnp.bfloat16)
        p2_ref[:] = jax.lax.dot_general(
            jnp.ones((1, _D), jnp.float32), p * p,
            (((1,), (1,)), ((), ())),
            preferred_element_type=jnp.float32)
        _compute(z_ref, pm2_ref, cross_a, z2_a)

    @pl.when(i % 2 == 1)
    def _odd():
        _compute(z_ref, pm2_ref, cross_b, z2_b)
        _epilogue(i, cross_a, z2_a, p2_ref, colmin_ref, rowacc_ref)

    @pl.when(jnp.logical_and(i % 2 == 0, jnp.logical_and(i > 0, i < _NB)))
    def _even():
        _compute(z_ref, pm2_ref, cross_a, z2_a)
        _epilogue(i, cross_b, z2_b, p2_ref, colmin_ref, rowacc_ref)

    @pl.when(i == _NB)
    def _last():
        _epilogue(i, cross_b, z2_b, p2_ref, colmin_ref, rowacc_ref)
        cm = jnp.sqrt(jnp.maximum(colmin_ref[:], 0.0))
        val = (_REG1 * (jnp.sum(rowacc_ref[:]) / _B)
               + _REG2 * (jnp.sum(cm) / _P))
        out_ref[...] = jnp.reshape(val, (1, 1))


def kernel(z, prototype_vectors):
    out = pl.pallas_call(
        _loss_body,
        grid=(_NB + 1,),
        in_specs=[
            pl.BlockSpec((_BZ, _D), lambda i: (jnp.minimum(i, _NB - 1), 0)),
            pl.BlockSpec((_P, _D), lambda i: (0, 0)),
        ],
        out_specs=pl.BlockSpec((1, 1), lambda i: (0, 0)),
        out_shape=jax.ShapeDtypeStruct((1, 1), jnp.float32),
        scratch_shapes=[
            pltpu.VMEM((_P, _D), jnp.bfloat16),
            pltpu.VMEM((1, _P), jnp.float32),
            pltpu.VMEM((_BZ, _P), jnp.float32),
            pltpu.VMEM((_BZ, _P), jnp.float32),
            pltpu.VMEM((_BZ, 1), jnp.float32),
            pltpu.VMEM((_BZ, 1), jnp.float32),
            pltpu.VMEM((1, _P), jnp.float32),
            pltpu.VMEM((_BZ, 1), jnp.float32),
        ],
    )(z, prototype_vectors)
    return out[0, 0]


# speedup vs baseline: 1.1661x; 1.1661x over previous
import jax
import jax.numpy as jnp
from jax.experimental import pallas as pl
from jax.experimental.pallas import tpu as pltpu

_B = 16384
_P = 1024
_D = 128
_K = _D + 2
_BZ = 1024
_NB = _B // _BZ
_REG1 = 0.05
_REG2 = 0.05


def _loss_body(z_ref, p_ref, out_ref, paug_ref, colmin_ref, rowacc_ref):
    i = pl.program_id(0)

    @pl.when(i == 0)
    def _prep():
        p = p_ref[:]
        p2 = jnp.sum(p * p, axis=1, keepdims=True)
        paug_ref[:, :_D] = (-2.0 * p).astype(jnp.bfloat16)
        paug_ref[:, _D:_D + 1] = jnp.ones((_P, 1), jnp.bfloat16)
        paug_ref[:, _D + 1:] = p2.astype(jnp.bfloat16)

    zb = z_ref[:]
    z2 = jnp.sum(zb * zb, axis=1, keepdims=True)
    zaug = jnp.concatenate(
        [zb.astype(jnp.bfloat16), z2.astype(jnp.bfloat16),
         jnp.ones((_BZ, 1), jnp.bfloat16)], axis=1)
    d2 = jax.lax.dot_general(
        zaug, paug_ref[:], (((1,), (1,)), ((), ())),
        preferred_element_type=jnp.float32)
    rowmin = jnp.min(d2, axis=1, keepdims=True)
    rowpart = jnp.sqrt(jnp.maximum(rowmin, 0.0))
    colpart = jnp.min(d2, axis=0, keepdims=True)

    @pl.when(i == 0)
    def _init():
        rowacc_ref[:] = rowpart
        colmin_ref[:] = colpart

    @pl.when(i > 0)
    def _accum():
        rowacc_ref[:] = rowacc_ref[:] + rowpart
        colmin_ref[:] = jnp.minimum(colmin_ref[:], colpart)

    @pl.when(i == _NB - 1)
    def _finish():
        cm = jnp.sqrt(jnp.maximum(colmin_ref[:], 0.0))
        val = (_REG1 * (jnp.sum(rowacc_ref[:]) / _B)
               + _REG2 * (jnp.sum(cm) / _P))
        out_ref[...] = jnp.reshape(val, (1, 1))


def kernel(z, prototype_vectors):
    out = pl.pallas_call(
        _loss_body,
        grid=(_NB,),
        in_specs=[
            pl.BlockSpec((_BZ, _D), lambda i: (i, 0)),
            pl.BlockSpec((_P, _D), lambda i: (0, 0)),
        ],
        out_specs=pl.BlockSpec((1, 1), lambda i: (0, 0)),
        out_shape=jax.ShapeDtypeStruct((1, 1), jnp.float32),
        scratch_shapes=[
            pltpu.VMEM((_P, _K), jnp.bfloat16),
            pltpu.VMEM((1, _P), jnp.float32),
            pltpu.VMEM((_BZ, 1), jnp.float32),
        ],
    )(z, prototype_vectors)
    return out[0, 0]
